# baseline (speedup 1.0000x reference)
import functools

import jax
import jax.numpy as jnp
from jax import lax
from jax.experimental import pallas as pl
from jax.experimental.pallas import tpu as pltpu
from jax.experimental.pallas import tpu_sc as plsc

B = 16384
L = 20
D = 128
TD = 3 * D
H = 1024
OUT = 1024

NC = 2
NS = 16
NW = NC * NS

SPW = B // NW
CHUNK = 32
NCH = SPW // CHUNK
CL = CHUNK * L
GW = 128
NSUB = CL // GW

_MESH = plsc.VectorSubcoreMesh(core_axis_name="c", subcore_axis_name="s")


@functools.partial(
    pl.kernel,
    mesh=_MESH,
    compiler_params=pltpu.CompilerParams(use_tc_tiling_on_sc=False,
                                         needs_layout_passes=False),
    out_type=jax.ShapeDtypeStruct((B, TD // 2), jnp.float32),
    scratch_types=[
        pltpu.VMEM((3 * SPW * L,), jnp.int32),
        pltpu.VMEM((CL, D // 2), jnp.float32),
        pltpu.VMEM((CHUNK, TD // 2), jnp.float32),
        pltpu.SemaphoreType.DMA((NSUB,)),
    ],
)
def _pool(xw, x2, x3, tw, t2, t3, out, idx_v, rows_v, acc_v, sems):
    wid = lax.axis_index("s") * NC + lax.axis_index("c")
    base = wid * SPW

    for t, xh in enumerate((xw, x2, x3)):
        pltpu.sync_copy(xh.at[pl.ds(wid * SPW * L, SPW * L)],
                        idx_v.at[pl.ds(t * SPW * L, SPW * L)])

    def chunk_body(c, carry):
        s0 = base + c * CHUNK

        for t, th in enumerate((tw, t2, t3)):
            for j in range(NSUB):
                pltpu.async_copy(
                    th.at[idx_v.at[pl.ds((t * NCH + c) * CL + j * GW, GW)]],
                    rows_v.at[pl.ds(j * GW, GW)], sems.at[j])

            def samp_body(s, carry2):
                r = s * L
                for g in range(D // 32):
                    col = pl.ds(g * 16, 16)
                    acc_a, acc_b = plsc.unpack(
                        plsc.bitcast(rows_v[r, col], jnp.bfloat16),
                        format=plsc.PackFormat.INTERLEAVED)
                    for l in range(1, L):
                        a, b = plsc.unpack(
                            plsc.bitcast(rows_v[r + l, col], jnp.bfloat16),
                            format=plsc.PackFormat.INTERLEAVED)
                        acc_a = acc_a + a
                        acc_b = acc_b + b
                    acc_v[s, pl.ds(t * (D // 2) + g * 16, 16)] = plsc.bitcast(
                        plsc.pack(acc_a * (1.0 / L), acc_b * (1.0 / L),
                                  format=plsc.PackFormat.INTERLEAVED),
                        jnp.float32)
                return carry2

            def drain_body(j, carry2):
                pltpu.make_async_copy(th.at[pl.ds(0, GW)],
                                      rows_v.at[pl.ds(0, GW)],
                                      sems.at[j]).wait()
                lo = j * GW // L
                hi = jnp.minimum((j + 1) * GW // L, CHUNK)
                lax.fori_loop(lo, hi, samp_body, 0)
                return carry2

            lax.fori_loop(0, NSUB, drain_body, 0)

        pltpu.sync_copy(acc_v, out.at[pl.ds(s0, CHUNK)])
        return carry

    lax.fori_loop(0, NCH, chunk_body, 0)


BM = 512


def _mlp_body(p_ref, w1_ref, b1_ref, w2_ref, b2_ref, o_ref):
    h = jnp.dot(p_ref[...], w1_ref[...], preferred_element_type=jnp.float32)
    h = jnp.maximum(h + b1_ref[...], 0.0)
    o_ref[...] = (
        jnp.dot(h.astype(jnp.bfloat16), w2_ref[...],
                preferred_element_type=jnp.float32)
        + b2_ref[...]
    )


_mlp = pl.pallas_call(
    _mlp_body,
    grid=(B // BM,),
    in_specs=[
        pl.BlockSpec((BM, TD), lambda i: (i, 0)),
        pl.BlockSpec((TD, H), lambda i: (0, 0)),
        pl.BlockSpec((1, H), lambda i: (0, 0)),
        pl.BlockSpec((H, OUT), lambda i: (0, 0)),
        pl.BlockSpec((1, OUT), lambda i: (0, 0)),
    ],
    out_specs=pl.BlockSpec((BM, OUT), lambda i: (i, 0)),
    out_shape=jax.ShapeDtypeStruct((B, OUT), jnp.float32),
)


def _pack_table(t):
    bf = t.astype(jnp.bfloat16).reshape(t.shape[0], t.shape[1] // 2, 2)
    return jax.lax.bitcast_convert_type(bf, jnp.float32)


def kernel(x, emb_word, emb_ngram2, emb_ngram3, W1, b1, W2, b2):
    bf = jnp.bfloat16
    pooled_packed = _pool(x[0].reshape(B * L), x[2].reshape(B * L),
                          x[3].reshape(B * L),
                          _pack_table(emb_word), _pack_table(emb_ngram2),
                          _pack_table(emb_ngram3))
    pooled = jax.lax.bitcast_convert_type(pooled_packed, bf).reshape(B, TD)
    return _mlp(pooled, W1.astype(bf), b1.reshape(1, H), W2.astype(bf),
                b2.reshape(1, OUT))


# speedup vs baseline: 2.9298x; 2.9298x over previous
import functools

import jax
import jax.numpy as jnp
from jax import lax
from jax.experimental import pallas as pl
from jax.experimental.pallas import tpu as pltpu
from jax.experimental.pallas import tpu_sc as plsc

B = 16384
L = 20
D = 128
TD = 3 * D
H = 1024
OUT = 1024

NC = 2
NS = 16
NW = NC * NS

SPW = B // NW
CHUNK = 32
NCH = SPW // CHUNK
CL = CHUNK * L
GW = 128
NSUB = CL // GW

_MESH = plsc.VectorSubcoreMesh(core_axis_name="c", subcore_axis_name="s")


@functools.partial(
    pl.kernel,
    mesh=_MESH,
    out_type=jax.ShapeDtypeStruct((B, TD), jnp.float32),
    scratch_types=[
        pltpu.VMEM((3 * SPW * L,), jnp.int32),
        pltpu.VMEM((CL, D), jnp.float32),
        pltpu.VMEM((CHUNK, TD), jnp.float32),
        pltpu.SemaphoreType.DMA((NSUB,)),
    ],
)
def _pool(xw, x2, x3, tw, t2, t3, out, idx_v, rows_v, acc_v, sems):
    wid = lax.axis_index("s") * NC + lax.axis_index("c")

    for t, xh in enumerate((xw, x2, x3)):
        pltpu.sync_copy(xh.at[pl.ds(wid * SPW * L, SPW * L)],
                        idx_v.at[pl.ds(t * SPW * L, SPW * L)])

    def chunk_body(c, carry):
        s0 = wid * SPW + c * CHUNK

        for t, th in enumerate((tw, t2, t3)):
            for j in range(NSUB):
                pltpu.async_copy(
                    th.at[idx_v.at[pl.ds((t * NCH + c) * CL + j * GW, GW)]],
                    rows_v.at[pl.ds(j * GW, GW)], sems.at[j])

            def samp_body(s, carry2):
                r = s * L
                for v in range(D // 16):
                    col = pl.ds(v * 16, 16)
                    accv = rows_v[r, col]
                    for l in range(1, L):
                        accv = accv + rows_v[r + l, col]
                    acc_v[s, pl.ds(t * D + v * 16, 16)] = accv * (1.0 / L)
                return carry2

            def drain_body(j, carry2):
                pltpu.make_async_copy(th.at[pl.ds(0, GW)],
                                      rows_v.at[pl.ds(0, GW)],
                                      sems.at[j]).wait()
                lo = j * GW // L
                hi = jnp.minimum((j + 1) * GW // L, CHUNK)
                lax.fori_loop(lo, hi, samp_body, 0)
                return carry2

            lax.fori_loop(0, NSUB, drain_body, 0)

        pltpu.sync_copy(acc_v, out.at[pl.ds(s0, CHUNK)])
        return carry

    lax.fori_loop(0, NCH, chunk_body, 0)


BM = 512


def _mlp_body(p_ref, w1_ref, b1_ref, w2_ref, b2_ref, o_ref):
    p = p_ref[...].astype(jnp.bfloat16)
    h = jnp.dot(p, w1_ref[...], preferred_element_type=jnp.float32)
    h = jnp.maximum(h + b1_ref[...], 0.0)
    o_ref[...] = (
        jnp.dot(h.astype(jnp.bfloat16), w2_ref[...],
                preferred_element_type=jnp.float32)
        + b2_ref[...]
    )


_mlp = pl.pallas_call(
    _mlp_body,
    grid=(B // BM,),
    in_specs=[
        pl.BlockSpec((BM, TD), lambda i: (i, 0)),
        pl.BlockSpec((TD, H), lambda i: (0, 0)),
        pl.BlockSpec((1, H), lambda i: (0, 0)),
        pl.BlockSpec((H, OUT), lambda i: (0, 0)),
        pl.BlockSpec((1, OUT), lambda i: (0, 0)),
    ],
    out_specs=pl.BlockSpec((BM, OUT), lambda i: (i, 0)),
    out_shape=jax.ShapeDtypeStruct((B, OUT), jnp.float32),
)


def kernel(x, emb_word, emb_ngram2, emb_ngram3, W1, b1, W2, b2):
    bf = jnp.bfloat16
    pooled = _pool(x[0].reshape(B * L), x[2].reshape(B * L),
                   x[3].reshape(B * L),
                   emb_word, emb_ngram2, emb_ngram3)
    return _mlp(pooled, W1.astype(bf), b1.reshape(1, H), W2.astype(bf),
                b2.reshape(1, OUT))


# speedup vs baseline: 3.8435x; 1.3119x over previous
import functools

import jax
import jax.numpy as jnp
from jax import lax
from jax.experimental import pallas as pl
from jax.experimental.pallas import tpu as pltpu
from jax.experimental.pallas import tpu_sc as plsc

B = 16384
L = 20
D = 128
TD = 3 * D
H = 1024
OUT = 1024

NC = 2
NS = 16
NW = NC * NS

SPW = B // NW
CHUNK = 32
NCH = SPW // CHUNK
CL = CHUNK * L
HS = CHUNK // 2
HL = HS * L
GW = 64
NSUB = HL // GW

_MESH = plsc.VectorSubcoreMesh(core_axis_name="c", subcore_axis_name="s")


@functools.partial(
    pl.kernel,
    mesh=_MESH,
    out_type=jax.ShapeDtypeStruct((B, TD), jnp.float32),
    scratch_types=[
        pltpu.VMEM((3 * SPW * L,), jnp.int32),
        pltpu.VMEM((HL, D), jnp.float32),
        pltpu.VMEM((HL, D), jnp.float32),
        pltpu.VMEM((CHUNK, TD), jnp.float32),
        pltpu.SemaphoreType.DMA((NSUB,)),
        pltpu.SemaphoreType.DMA((NSUB,)),
    ],
)
def _pool(xw, x2, x3, tw, t2, t3, out, idx_v, rows_a, rows_b, acc_v,
          sems_a, sems_b):
    wid = lax.axis_index("s") * NC + lax.axis_index("c")
    tabs = (tw, t2, t3)
    bufs = (rows_a, rows_b)
    sems = (sems_a, sems_b)

    for t, xh in enumerate((xw, x2, x3)):
        pltpu.sync_copy(xh.at[pl.ds(wid * SPW * L, SPW * L)],
                        idx_v.at[pl.ds(t * SPW * L, SPW * L)])

    def fire(c, t, h):
        for j in range(NSUB):
            pltpu.async_copy(
                tabs[t].at[idx_v.at[pl.ds(
                    (t * NCH + c) * CL + h * HL + j * GW, GW)]],
                bufs[h].at[pl.ds(j * GW, GW)], sems[h].at[j])

    def drain(c, t, h):
        rows_v = bufs[h]

        def samp_body(s, carry2):
            r = s * L
            for v in range(D // 16):
                col = pl.ds(v * 16, 16)
                accv = rows_v[r, col]
                for l in range(1, L):
                    accv = accv + rows_v[r + l, col]
                acc_v[h * HS + s, pl.ds(t * D + v * 16, 16)] = (
                    accv * (1.0 / L))
            return carry2

        def drain_body(j, carry2):
            pltpu.make_async_copy(tabs[t].at[pl.ds(0, GW)],
                                  rows_v.at[pl.ds(0, GW)],
                                  sems[h].at[j]).wait()
            lo = j * GW // L
            hi = jnp.minimum((j + 1) * GW // L, HS)
            lax.fori_loop(lo, hi, samp_body, 0)
            return carry2

        lax.fori_loop(0, NSUB, drain_body, 0)

    fire(0, 0, 0)

    def chunk_body(c, carry):
        units = [(t, h) for t in range(3) for h in range(2)]
        for u, (t, h) in enumerate(units):
            if u + 1 < len(units):
                tn, hn = units[u + 1]
                fire(c, tn, hn)
            else:
                @pl.when(c + 1 < NCH)
                def _():
                    fire(c + 1, 0, 0)
            drain(c, t, h)

        pltpu.sync_copy(acc_v,
                        out.at[pl.ds(wid * SPW + c * CHUNK, CHUNK)])
        return carry

    lax.fori_loop(0, NCH, chunk_body, 0)


BM = 512


def _mlp_body(p_ref, w1_ref, b1_ref, w2_ref, b2_ref, o_ref):
    p = p_ref[...].astype(jnp.bfloat16)
    h = jnp.dot(p, w1_ref[...], preferred_element_type=jnp.float32)
    h = jnp.maximum(h + b1_ref[...], 0.0)
    o_ref[...] = (
        jnp.dot(h.astype(jnp.bfloat16), w2_ref[...],
                preferred_element_type=jnp.float32)
        + b2_ref[...]
    )


_mlp = pl.pallas_call(
    _mlp_body,
    grid=(B // BM,),
    in_specs=[
        pl.BlockSpec((BM, TD), lambda i: (i, 0)),
        pl.BlockSpec((TD, H), lambda i: (0, 0)),
        pl.BlockSpec((1, H), lambda i: (0, 0)),
        pl.BlockSpec((H, OUT), lambda i: (0, 0)),
        pl.BlockSpec((1, OUT), lambda i: (0, 0)),
    ],
    out_specs=pl.BlockSpec((BM, OUT), lambda i: (i, 0)),
    out_shape=jax.ShapeDtypeStruct((B, OUT), jnp.float32),
)


def kernel(x, emb_word, emb_ngram2, emb_ngram3, W1, b1, W2, b2):
    bf = jnp.bfloat16
    pooled = _pool(x[0].reshape(B * L), x[2].reshape(B * L),
                   x[3].reshape(B * L),
                   emb_word, emb_ngram2, emb_ngram3)
    return _mlp(pooled, W1.astype(bf), b1.reshape(1, H), W2.astype(bf),
                b2.reshape(1, OUT))


# speedup vs baseline: 3.8913x; 1.0124x over previous
import functools

import jax
import jax.numpy as jnp
from jax import lax
from jax.experimental import pallas as pl
from jax.experimental.pallas import tpu as pltpu
from jax.experimental.pallas import tpu_sc as plsc

B = 16384
L = 20
D = 128
TD = 3 * D
H = 1024
OUT = 1024

NC = 2
NS = 16
NW = NC * NS

SPW = B // NW
CHUNK = 32
NCH = SPW // CHUNK
CL = CHUNK * L
HS = CHUNK // 2
HL = HS * L
GW = 64
NSUB = HL // GW

_MESH = plsc.VectorSubcoreMesh(core_axis_name="c", subcore_axis_name="s")


@functools.partial(
    pl.kernel,
    mesh=_MESH,
    out_type=jax.ShapeDtypeStruct((B, TD), jnp.float32),
    scratch_types=[
        pltpu.VMEM((3 * SPW * L,), jnp.int32),
        pltpu.VMEM((HL, D), jnp.float32),
        pltpu.VMEM((HL, D), jnp.float32),
        pltpu.VMEM((CHUNK, TD), jnp.float32),
        pltpu.SemaphoreType.DMA((NSUB,)),
        pltpu.SemaphoreType.DMA((NSUB,)),
        pltpu.SemaphoreType.DMA,
    ],
)
def _pool(xf, tw, t2, t3, out, idx_v, rows_a, rows_b, acc_v,
          sems_a, sems_b, sem_idx):
    wid = lax.axis_index("s") * NC + lax.axis_index("c")
    tabs = (tw, t2, t3)
    bufs = (rows_a, rows_b)
    sems = (sems_a, sems_b)

    pltpu.sync_copy(xf.at[pl.ds(wid * SPW * L, SPW * L)],
                    idx_v.at[pl.ds(0, SPW * L)])
    idx_cps = [
        pltpu.async_copy(
            xf.at[pl.ds(tsel * B * L + wid * SPW * L, SPW * L)],
            idx_v.at[pl.ds(t * SPW * L, SPW * L)], sem_idx)
        for t, tsel in ((1, 2), (2, 3))
    ]

    def fire(c, t, h):
        for j in range(NSUB):
            pltpu.async_copy(
                tabs[t].at[idx_v.at[pl.ds(
                    (t * NCH + c) * CL + h * HL + j * GW, GW)]],
                bufs[h].at[pl.ds(j * GW, GW)], sems[h].at[j])

    def drain(c, t, h):
        rows_v = bufs[h]

        def samp_body(s, carry2):
            r = s * L
            for v in range(D // 16):
                col = pl.ds(v * 16, 16)
                accv = rows_v[r, col]
                for l in range(1, L):
                    accv = accv + rows_v[r + l, col]
                acc_v[h * HS + s, pl.ds(t * D + v * 16, 16)] = (
                    accv * (1.0 / L))
            return carry2

        def drain_body(j, carry2):
            pltpu.make_async_copy(tabs[t].at[pl.ds(0, GW)],
                                  rows_v.at[pl.ds(0, GW)],
                                  sems[h].at[j]).wait()
            lo = j * GW // L
            hi = jnp.minimum((j + 1) * GW // L, HS)
            lax.fori_loop(lo, hi, samp_body, 0)
            return carry2

        lax.fori_loop(0, NSUB, drain_body, 0)

    fire(0, 0, 0)
    for cp in idx_cps:
        cp.wait()

    def chunk_body(c, carry):
        units = [(t, h) for t in range(3) for h in range(2)]
        for u, (t, h) in enumerate(units):
            if u + 1 < len(units):
                tn, hn = units[u + 1]
                fire(c, tn, hn)
            else:
                @pl.when(c + 1 < NCH)
                def _():
                    fire(c + 1, 0, 0)
            drain(c, t, h)

        pltpu.sync_copy(acc_v,
                        out.at[pl.ds(wid * SPW + c * CHUNK, CHUNK)])
        return carry

    lax.fori_loop(0, NCH, chunk_body, 0)


BM = 512


def _mlp_body(p_ref, w1_ref, b1_ref, w2_ref, b2_ref, o_ref):
    p = p_ref[...].astype(jnp.bfloat16)
    h = jnp.dot(p, w1_ref[...], preferred_element_type=jnp.float32)
    h = jnp.maximum(h + b1_ref[...], 0.0)
    o_ref[...] = (
        jnp.dot(h.astype(jnp.bfloat16), w2_ref[...],
                preferred_element_type=jnp.float32)
        + b2_ref[...]
    )


_mlp = pl.pallas_call(
    _mlp_body,
    grid=(B // BM,),
    in_specs=[
        pl.BlockSpec((BM, TD), lambda i: (i, 0)),
        pl.BlockSpec((TD, H), lambda i: (0, 0)),
        pl.BlockSpec((1, H), lambda i: (0, 0)),
        pl.BlockSpec((H, OUT), lambda i: (0, 0)),
        pl.BlockSpec((1, OUT), lambda i: (0, 0)),
    ],
    out_specs=pl.BlockSpec((BM, OUT), lambda i: (i, 0)),
    out_shape=jax.ShapeDtypeStruct((B, OUT), jnp.float32),
)


def kernel(x, emb_word, emb_ngram2, emb_ngram3, W1, b1, W2, b2):
    bf = jnp.bfloat16
    pooled = _pool(x.reshape(4 * B * L), emb_word, emb_ngram2, emb_ngram3)
    return _mlp(pooled, W1.astype(bf), b1.reshape(1, H), W2.astype(bf),
                b2.reshape(1, OUT))


# speedup vs baseline: 3.9265x; 1.0090x over previous
import functools

import jax
import jax.numpy as jnp
from jax import lax
from jax.experimental import pallas as pl
from jax.experimental.pallas import tpu as pltpu
from jax.experimental.pallas import tpu_sc as plsc

B = 16384
L = 20
D = 128
TD = 3 * D
H = 1024
OUT = 1024

NC = 2
NS = 16
NW = NC * NS

SPW = B // NW
CHUNK = 32
NCH = SPW // CHUNK
CL = CHUNK * L
HS = CHUNK // 2
HL = HS * L
GW = 64
NSUB = HL // GW

_MESH = plsc.VectorSubcoreMesh(core_axis_name="c", subcore_axis_name="s")


@functools.partial(
    pl.kernel,
    mesh=_MESH,
    out_type=jax.ShapeDtypeStruct((B, TD), jnp.float32),
    scratch_types=[
        pltpu.VMEM((3 * SPW * L,), jnp.int32),
        pltpu.VMEM((HL, D), jnp.float32),
        pltpu.VMEM((HL, D), jnp.float32),
        pltpu.VMEM((CHUNK, TD), jnp.float32),
        pltpu.SemaphoreType.DMA((NSUB,)),
        pltpu.SemaphoreType.DMA((NSUB,)),
        pltpu.SemaphoreType.DMA,
    ],
)
def _pool(xf, tw, t2, t3, out, idx_v, rows_a, rows_b, acc_v,
          sems_a, sems_b, sem_idx):
    wid = lax.axis_index("s") * NC + lax.axis_index("c")
    tabs = (tw, t2, t3)
    bufs = (rows_a, rows_b)
    sems = (sems_a, sems_b)

    pltpu.sync_copy(xf.at[pl.ds(wid * SPW * L, SPW * L)],
                    idx_v.at[pl.ds(0, SPW * L)])
    idx_cps = [
        pltpu.async_copy(
            xf.at[pl.ds(tsel * B * L + wid * SPW * L, SPW * L)],
            idx_v.at[pl.ds(t * SPW * L, SPW * L)], sem_idx)
        for t, tsel in ((1, 2), (2, 3))
    ]

    def fire(c, t, h):
        for j in range(NSUB):
            pltpu.async_copy(
                tabs[t].at[idx_v.at[pl.ds(
                    (t * NCH + c) * CL + h * HL + j * GW, GW)]],
                bufs[h].at[pl.ds(j * GW, GW)], sems[h].at[j])

    def drain(c, t, h):
        rows_v = bufs[h]

        def samp_body(s, carry2):
            r = s * L
            for v in range(D // 16):
                col = pl.ds(v * 16, 16)
                accv = rows_v[r, col]
                for l in range(1, L):
                    accv = accv + rows_v[r + l, col]
                acc_v[h * HS + s, pl.ds(t * D + v * 16, 16)] = (
                    accv * (1.0 / L))
            return carry2

        def drain_body(j, carry2):
            pltpu.make_async_copy(tabs[t].at[pl.ds(0, GW)],
                                  rows_v.at[pl.ds(0, GW)],
                                  sems[h].at[j]).wait()
            lo = j * GW // L
            hi = jnp.minimum((j + 1) * GW // L, HS)
            lax.fori_loop(lo, hi, samp_body, 0)
            return carry2

        lax.fori_loop(0, NSUB, drain_body, 0)

    fire(0, 0, 0)
    for cp in idx_cps:
        cp.wait()

    def chunk_body(c, carry):
        units = [(t, h) for t in range(3) for h in range(2)]
        for u, (t, h) in enumerate(units):
            if u + 1 < len(units):
                tn, hn = units[u + 1]
                fire(c, tn, hn)
            else:
                @pl.when(c + 1 < NCH)
                def _():
                    fire(c + 1, 0, 0)
            drain(c, t, h)

        pltpu.sync_copy(acc_v,
                        out.at[pl.ds(wid * SPW + c * CHUNK, CHUNK)])
        return carry

    lax.fori_loop(0, NCH, chunk_body, 0)


BM = 1024


def _mlp_body(p_ref, w1_ref, b1_ref, w2_ref, b2_ref, o_ref):
    p = p_ref[...].astype(jnp.bfloat16)
    h = jnp.dot(p, w1_ref[...], preferred_element_type=jnp.float32)
    h = jnp.maximum(h + b1_ref[...], 0.0)
    o_ref[...] = (
        jnp.dot(h.astype(jnp.bfloat16), w2_ref[...],
                preferred_element_type=jnp.float32)
        + b2_ref[...]
    )


_mlp = pl.pallas_call(
    _mlp_body,
    grid=(B // BM,),
    in_specs=[
        pl.BlockSpec((BM, TD), lambda i: (i, 0)),
        pl.BlockSpec((TD, H), lambda i: (0, 0)),
        pl.BlockSpec((1, H), lambda i: (0, 0)),
        pl.BlockSpec((H, OUT), lambda i: (0, 0)),
        pl.BlockSpec((1, OUT), lambda i: (0, 0)),
    ],
    out_specs=pl.BlockSpec((BM, OUT), lambda i: (i, 0)),
    out_shape=jax.ShapeDtypeStruct((B, OUT), jnp.float32),
)


def kernel(x, emb_word, emb_ngram2, emb_ngram3, W1, b1, W2, b2):
    bf = jnp.bfloat16
    pooled = _pool(x.reshape(4 * B * L), emb_word, emb_ngram2, emb_ngram3)
    return _mlp(pooled, W1.astype(bf), b1.reshape(1, H), W2.astype(bf),
                b2.reshape(1, OUT))
